# Initial kernel scaffold; baseline (speedup 1.0000x reference)
import jax
import jax.numpy as jnp
from jax.experimental import pallas as pl


def kernel(batch, positions, field, matrix):
    raise NotImplementedError("write your pallas kernel here")


# speedup vs baseline: 2.0307x; 2.0307x over previous
import functools

import jax
import jax.numpy as jnp
from jax import lax
from jax.experimental import pallas as pl
from jax.experimental.pallas import tpu as pltpu
from jax.experimental.pallas import tpu_sc as plsc

_NC = 2
_NS = 16
_NW = _NC * _NS
_CHUNK = 128


def _project_body(field_ref, matrix_ref, t_ref):
    m = matrix_ref[...]
    mp = jnp.concatenate(
        [m[:, 0:1], m[:, 2:3], m[:, 3:4], m[:, 1:2]], axis=1
    )
    t_ref[...] = lax.dot_general(
        field_ref[...], mp, (((1,), (1,)), ((), ())),
        preferred_element_type=jnp.float32,
    )


@functools.cache
def _make_project(g, p):
    return pl.pallas_call(
        _project_body,
        out_shape=jax.ShapeDtypeStruct((g, p), jnp.float32),
    )


@functools.cache
def _make_gather(n_pad, p):
    bpw = n_pad // _NW
    ch = bpw // _CHUNK
    mesh = plsc.VectorSubcoreMesh(
        core_axis_name="c", subcore_axis_name="s",
        num_cores=_NC, num_subcores=_NS,
    )

    @functools.partial(
        pl.kernel,
        out_type=jax.ShapeDtypeStruct((n_pad, p), jnp.float32),
        mesh=mesh,
        scratch_types=[
            pltpu.VMEM((bpw,), jnp.int32),
            pltpu.VMEM((bpw, p), jnp.float32),
            pltpu.SemaphoreType.DMA,
        ],
        compiler_params=pltpu.CompilerParams(use_tc_tiling_on_sc=False),
    )
    def gather(t_hbm, idx_hbm, out_hbm, idx_v, rows_v, sem):
        wid = lax.axis_index("s") * _NC + lax.axis_index("c")
        base = wid * bpw
        pltpu.sync_copy(idx_hbm.at[pl.ds(base, bpw)], idx_v)
        copies = [
            pltpu.async_copy(
                t_hbm.at[idx_v.at[pl.ds(j * _CHUNK, _CHUNK)]],
                rows_v.at[pl.ds(j * _CHUNK, _CHUNK)],
                sem,
            )
            for j in range(ch)
        ]
        for c in copies:
            c.wait()
        pltpu.sync_copy(rows_v, out_hbm.at[pl.ds(base, bpw)])

    return gather


def kernel(batch, positions, field, matrix):
    del positions
    n = batch.shape[0]
    g = field.shape[0]
    p = matrix.shape[0]

    t = _make_project(g, p)(field, matrix)

    per_w = _CHUNK * _NW
    n_pad = -(-n // per_w) * per_w
    idx = batch.astype(jnp.int32)
    if n_pad != n:
        idx = jnp.concatenate([idx, jnp.zeros((n_pad - n,), jnp.int32)])

    out = _make_gather(n_pad, p)(t, idx)
    return out[:n]


# speedup vs baseline: 2.8029x; 1.3802x over previous
import functools

import jax
import jax.numpy as jnp
from jax import lax
from jax.experimental import pallas as pl
from jax.experimental.pallas import tpu as pltpu
from jax.experimental.pallas import tpu_sc as plsc

_NC = 2
_NS = 16
_NW = _NC * _NS
_CHUNK = 128


def _project_body(field_ref, matrix_ref, t_ref):
    m = matrix_ref[...]
    mp = jnp.concatenate(
        [m[:, 0:1], m[:, 2:3], m[:, 3:4], m[:, 1:2]], axis=1
    )
    t_ref[...] = lax.dot_general(
        field_ref[...], mp, (((1,), (1,)), ((), ())),
        preferred_element_type=jnp.float32,
    )


@functools.cache
def _make_project(g, p):
    return pl.pallas_call(
        _project_body,
        out_shape=jax.ShapeDtypeStruct((g, p), jnp.float32),
    )


def _chunk_sizes(rows):
    full, rem = divmod(rows, _CHUNK)
    return [_CHUNK] * full + ([rem] if rem else [])


@functools.cache
def _make_gather(n, p):
    bpw = (-(-n // _NW) + 7) // 8 * 8
    tail = n - (_NW - 1) * bpw
    assert 0 < tail <= bpw
    mesh = plsc.VectorSubcoreMesh(
        core_axis_name="c", subcore_axis_name="s",
        num_cores=_NC, num_subcores=_NS,
    )

    @functools.partial(
        pl.kernel,
        out_type=jax.ShapeDtypeStruct((n, p), jnp.float32),
        mesh=mesh,
        scratch_types=[
            pltpu.VMEM((bpw,), jnp.int32),
            pltpu.VMEM((bpw, p), jnp.float32),
            pltpu.SemaphoreType.DMA,
        ],
        compiler_params=pltpu.CompilerParams(use_tc_tiling_on_sc=False),
    )
    def gather(t_hbm, idx_hbm, out_hbm, idx_v, rows_v, sem):
        wid = lax.axis_index("s") * _NC + lax.axis_index("c")
        base = wid * bpw

        def body(rows):
            pltpu.sync_copy(idx_hbm.at[pl.ds(base, rows)],
                            idx_v.at[pl.ds(0, rows)])
            copies = []
            off = 0
            for sz in _chunk_sizes(rows):
                copies.append(pltpu.async_copy(
                    t_hbm.at[idx_v.at[pl.ds(off, sz)]],
                    rows_v.at[pl.ds(off, sz)],
                    sem,
                ))
                off += sz
            for c in copies:
                c.wait()
            pltpu.sync_copy(rows_v.at[pl.ds(0, rows)],
                            out_hbm.at[pl.ds(base, rows)])

        if tail == bpw:
            body(bpw)
        else:
            @pl.when(wid < _NW - 1)
            def _main():
                body(bpw)

            @pl.when(wid == _NW - 1)
            def _tail():
                body(tail)

    return gather


def kernel(batch, positions, field, matrix):
    del positions
    n = batch.shape[0]
    g = field.shape[0]
    p = matrix.shape[0]

    t = _make_project(g, p)(field, matrix)

    idx = batch.astype(jnp.int32)
    return _make_gather(n, p)(t, idx)


# speedup vs baseline: 5.0626x; 1.8062x over previous
import functools

import jax
import jax.numpy as jnp
from jax import lax
from jax.experimental import pallas as pl
from jax.experimental.pallas import tpu as pltpu
from jax.experimental.pallas import tpu_sc as plsc

_NC = 2
_NS = 16
_NW = _NC * _NS
_L = 16


def _project_body(field_ref, matrix_ref, t_ref):
    m = matrix_ref[...]
    mp = jnp.concatenate(
        [m[:, 0:1], m[:, 2:3], m[:, 3:4], m[:, 1:2]], axis=1
    )
    t_ref[...] = lax.dot_general(
        mp, field_ref[...], (((1,), (1,)), ((), ())),
        preferred_element_type=jnp.float32,
    )


@functools.cache
def _make_project(g, p):
    return pl.pallas_call(
        _project_body,
        out_shape=jax.ShapeDtypeStruct((p, g), jnp.float32),
    )


@functools.cache
def _make_gather(n, g, p):
    assert n % _L == 0
    bpw = (-(-n // _NW) + _L - 1) // _L * _L
    tail = n - (_NW - 1) * bpw
    assert 0 < tail <= bpw and tail % _L == 0
    mesh = plsc.VectorSubcoreMesh(
        core_axis_name="c", subcore_axis_name="s",
        num_cores=_NC, num_subcores=_NS,
    )

    @functools.partial(
        pl.kernel,
        out_type=jax.ShapeDtypeStruct((p, n), jnp.float32),
        mesh=mesh,
        scratch_types=[
            pltpu.VMEM((p * g,), jnp.float32),
            pltpu.VMEM((bpw,), jnp.int32),
            pltpu.VMEM((p * bpw,), jnp.float32),
            pltpu.SemaphoreType.DMA,
        ],
        compiler_params=pltpu.CompilerParams(
            use_tc_tiling_on_sc=False, needs_layout_passes=False,
        ),
    )
    def gather(t_hbm, idx_hbm, out_hbm, t_v, idx_v, blk_v, sem):
        wid = lax.axis_index("s") * _NC + lax.axis_index("c")
        base = wid * bpw
        pltpu.sync_copy(t_hbm, t_v)

        def body(rows):
            pltpu.sync_copy(idx_hbm.at[pl.ds(base, rows)],
                            idx_v.at[pl.ds(0, rows)])

            def group(i, _):
                off = i * _L
                idx = idx_v[pl.ds(off, _L)]
                for pp in range(p):
                    vals = plsc.load_gather(t_v, [idx + pp * g])
                    blk_v[pl.ds(pp * bpw + off, _L)] = vals
                return 0

            lax.fori_loop(0, rows // _L, group, 0)
            copies = [
                pltpu.async_copy(
                    blk_v.at[pl.ds(pp * bpw, rows)],
                    out_hbm.at[pp, pl.ds(base, rows)],
                    sem,
                )
                for pp in range(p)
            ]
            for c in copies:
                c.wait()

        if tail == bpw:
            body(bpw)
        else:
            @pl.when(wid < _NW - 1)
            def _main():
                body(bpw)

            @pl.when(wid == _NW - 1)
            def _tail():
                body(tail)

    return gather


def kernel(batch, positions, field, matrix):
    del positions
    n = batch.shape[0]
    g = field.shape[0]
    p = matrix.shape[0]

    t = _make_project(g, p)(field, matrix)
    idx = batch.astype(jnp.int32)
    out_t = _make_gather(n, g, p)(t.reshape(-1), idx)
    return out_t.T


# speedup vs baseline: 6.5233x; 1.2885x over previous
import functools

import jax
import jax.numpy as jnp
from jax import lax
from jax.experimental import pallas as pl
from jax.experimental.pallas import tpu as pltpu
from jax.experimental.pallas import tpu_sc as plsc

_NC = 2
_NS = 16
_NW = _NC * _NS
_L = 16


def _project_body(field_ref, matrix_ref, t_ref):
    m = matrix_ref[...]
    mp = jnp.concatenate(
        [m[:, 0:1], m[:, 2:3], m[:, 3:4], m[:, 1:2]], axis=1
    )
    t_ref[...] = lax.dot_general(
        mp, field_ref[...], (((1,), (1,)), ((), ())),
        preferred_element_type=jnp.float32,
    )


@functools.cache
def _make_project(g, p):
    return pl.pallas_call(
        _project_body,
        out_shape=jax.ShapeDtypeStruct((p, g), jnp.float32),
    )


@functools.cache
def _make_gather(n, g, p):
    assert n % _L == 0
    bpw = (-(-n // _NW) + _L - 1) // _L * _L
    tail = n - (_NW - 1) * bpw
    assert 0 < tail <= bpw and tail % _L == 0
    mesh = plsc.VectorSubcoreMesh(
        core_axis_name="c", subcore_axis_name="s",
        num_cores=_NC, num_subcores=_NS,
    )

    @functools.partial(
        pl.kernel,
        out_type=jax.ShapeDtypeStruct((p, n), jnp.float32),
        mesh=mesh,
        scratch_types=[
            pltpu.VMEM((p * g,), jnp.float32),
            pltpu.VMEM((bpw,), jnp.int32),
            pltpu.VMEM((p * bpw,), jnp.float32),
            pltpu.SemaphoreType.DMA,
            pltpu.SemaphoreType.DMA,
        ],
        compiler_params=pltpu.CompilerParams(
            use_tc_tiling_on_sc=False, needs_layout_passes=False,
        ),
    )
    def gather(t_hbm, idx_hbm, out_hbm, t_v, idx_v, blk_v, sem, sem2):
        wid = lax.axis_index("s") * _NC + lax.axis_index("c")
        base = wid * bpw

        def body(rows):
            c_t = pltpu.async_copy(t_hbm, t_v, sem2)
            c_i = pltpu.async_copy(idx_hbm.at[pl.ds(base, rows)],
                                   idx_v.at[pl.ds(0, rows)], sem)
            c_t.wait()
            c_i.wait()

            half = rows // 2
            copies = []
            for h in range(2):
                hoff = h * half

                @functools.partial(
                    plsc.parallel_loop, hoff // _L, (hoff + half) // _L,
                    unroll=4,
                )
                def group(i):
                    off = i * _L
                    idx = idx_v[pl.ds(off, _L)]
                    for pp in range(p):
                        vals = plsc.load_gather(t_v, [idx + pp * g])
                        blk_v[pl.ds(pp * bpw + off, _L)] = vals

                copies += [
                    pltpu.async_copy(
                        blk_v.at[pl.ds(pp * bpw + hoff, half)],
                        out_hbm.at[pp, pl.ds(base + hoff, half)],
                        sem,
                    )
                    for pp in range(p)
                ]
            for c in copies:
                c.wait()

        if tail == bpw:
            body(bpw)
        else:
            @pl.when(wid < _NW - 1)
            def _main():
                body(bpw)

            @pl.when(wid == _NW - 1)
            def _tail():
                body(tail)

    return gather


def kernel(batch, positions, field, matrix):
    del positions
    n = batch.shape[0]
    g = field.shape[0]
    p = matrix.shape[0]

    t = _make_project(g, p)(field, matrix)
    idx = batch.astype(jnp.int32)
    out_t = _make_gather(n, g, p)(t.reshape(-1), idx)
    return out_t.T


# speedup vs baseline: 8.2147x; 1.2593x over previous
import functools

import jax
import jax.numpy as jnp
from jax import lax
from jax.experimental import pallas as pl
from jax.experimental.pallas import tpu as pltpu
from jax.experimental.pallas import tpu_sc as plsc

_NC = 2
_NS = 16
_NW = _NC * _NS
_L = 16


def _project_body(field_ref, matrix_ref, t_ref):
    m = matrix_ref[...]
    mp = jnp.concatenate(
        [m[:, 0:1], m[:, 2:3], m[:, 3:4], m[:, 1:2]], axis=1
    )
    t_ref[...] = lax.dot_general(
        mp, field_ref[...], (((1,), (1,)), ((), ())),
        preferred_element_type=jnp.float32,
    )


@functools.cache
def _make_project(g, p):
    return pl.pallas_call(
        _project_body,
        out_shape=jax.ShapeDtypeStruct((p, g), jnp.float32),
    )


_TILE = 128


@functools.cache
def _make_gather(n, g, p):
    assert n % _L == 0 and p % 8 == 0
    nt = -(-n // _TILE)
    tpw = -(-nt // _NW)
    bpw = tpw * _TILE
    tail_t = nt - (_NW - 1) * tpw
    assert 0 < tail_t <= tpw
    tail_rows = n - (_NW - 1) * bpw
    assert 0 < tail_rows <= tail_t * _TILE and tail_rows % _L == 0
    mesh = plsc.VectorSubcoreMesh(
        core_axis_name="c", subcore_axis_name="s",
        num_cores=_NC, num_subcores=_NS,
    )

    @functools.partial(
        pl.kernel,
        out_type=jax.ShapeDtypeStruct((p // 8, nt, 8, _TILE), jnp.float32),
        mesh=mesh,
        scratch_types=[
            pltpu.VMEM((p * g,), jnp.float32),
            pltpu.VMEM((bpw,), jnp.int32),
            pltpu.VMEM((p, bpw), jnp.float32),
            pltpu.SemaphoreType.DMA,
            pltpu.SemaphoreType.DMA,
        ],
        compiler_params=pltpu.CompilerParams(
            use_tc_tiling_on_sc=False, needs_layout_passes=False,
        ),
    )
    def gather(t_hbm, idx_hbm, out_hbm, t_v, idx_v, blk_v, sem, sem2):
        wid = lax.axis_index("s") * _NC + lax.axis_index("c")
        base = wid * bpw
        tbase = wid * tpw

        def body(rows, tiles):
            c_t = pltpu.async_copy(t_hbm, t_v, sem2)
            c_i = pltpu.async_copy(idx_hbm.at[pl.ds(base, rows)],
                                   idx_v.at[pl.ds(0, rows)], sem)
            c_t.wait()
            c_i.wait()

            t0 = tiles // 2
            g0 = min(t0 * (_TILE // _L), rows // _L)
            bounds = [(0, g0, 0, t0), (g0, rows // _L, t0, tiles)]
            copies = []
            for glo, ghi, tlo, thi in bounds:
                @functools.partial(
                    plsc.parallel_loop, glo, ghi, unroll=4,
                )
                def group(i):
                    off = i * _L
                    idx = idx_v[pl.ds(off, _L)]
                    for pp in range(p):
                        vals = plsc.load_gather(t_v, [idx + pp * g])
                        blk_v[pp, pl.ds(off, _L)] = vals

                copies += [
                    pltpu.async_copy(
                        blk_v.at[pl.ds(pt * 8, 8), pl.ds(j * _TILE, _TILE)],
                        out_hbm.at[pt, tbase + j],
                        sem,
                    )
                    for j in range(tlo, thi)
                    for pt in range(p // 8)
                ]
            for c in copies:
                c.wait()

        if tail_t == tpw and tail_rows == bpw:
            body(bpw, tpw)
        else:
            @pl.when(wid < _NW - 1)
            def _main():
                body(bpw, tpw)

            @pl.when(wid == _NW - 1)
            def _tail():
                body(tail_rows, tail_t)

    return gather


def kernel(batch, positions, field, matrix):
    del positions
    n = batch.shape[0]
    g = field.shape[0]
    p = matrix.shape[0]

    t = _make_project(g, p)(field, matrix)
    idx = batch.astype(jnp.int32)
    out4 = _make_gather(n, g, p)(t.reshape(-1), idx)
    nt = out4.shape[1]
    return out4.transpose(1, 3, 0, 2).reshape(nt * _TILE, p)[:n]
